# baseline (speedup 1.0000x reference)
import functools

import jax
import jax.numpy as jnp
from jax import lax
from jax.experimental import pallas as pl
from jax.experimental.pallas import tpu as pltpu
from jax.experimental.pallas import tpu_sc as plsc

VOCAB = 1000000
EMBED = 64
OUT = 4
SEQ = 200
BATCH = 4096

NC = 2
NS = 16
NW = NC * NS
BPW = BATCH // NW
CHUNK = SEQ // 2
CPW = BPW * 2
INV_SEQ = 1.0 / SEQ

_mesh = plsc.VectorSubcoreMesh(
    core_axis_name="c", subcore_axis_name="s", num_cores=NC, num_subcores=NS
)


def _reduce_chunk(buf, acc):
    def body(i, acc):
        a0, a1, a2, a3 = acc
        s = i * 4
        for k in range(4):
            a0 = a0 + buf[s + k, pl.ds(0, 16)]
            a1 = a1 + buf[s + k, pl.ds(16, 16)]
            a2 = a2 + buf[s + k, pl.ds(32, 16)]
            a3 = a3 + buf[s + k, pl.ds(48, 16)]
        return (a0, a1, a2, a3)

    return lax.fori_loop(0, CHUNK // 4, body, acc)


@functools.partial(
    pl.kernel,
    out_type=jax.ShapeDtypeStruct((BATCH, EMBED), jnp.float32),
    mesh=_mesh,
    scratch_types=[
        pltpu.VMEM((CPW, CHUNK), jnp.int32),
        pltpu.VMEM((4, CHUNK, EMBED), jnp.float32),
        pltpu.VMEM((BPW, EMBED), jnp.float32),
        pltpu.SemaphoreType.DMA,
        pltpu.SemaphoreType.DMA,
        pltpu.SemaphoreType.DMA,
        pltpu.SemaphoreType.DMA,
    ],
    compiler_params=pltpu.CompilerParams(use_tc_tiling_on_sc=False),
)
def _pooled_kernel(tt_hbm, table_hbm, pooled_hbm, idx_v, rows_v, pooled_v,
                   sem0, sem1, sem2, sem3):
    sems = (sem0, sem1, sem2, sem3)
    wid = lax.axis_index("s") * NC + lax.axis_index("c")
    cbase = wid * CPW
    bbase = wid * BPW

    pltpu.sync_copy(tt_hbm.at[pl.ds(cbase, CPW)], idx_v)

    def start(c, buf):
        pltpu.async_copy(table_hbm.at[idx_v.at[c]], rows_v.at[buf], sems[buf])

    def wait(buf):
        pltpu.make_async_copy(
            table_hbm.at[idx_v.at[0]], rows_v.at[buf], sems[buf]
        ).wait()

    for k in range(4):
        start(k, k)

    zeros = jnp.zeros((16,), jnp.float32)

    def do_batch(i, b, buf_a, buf_b):
        wait(buf_a)
        acc = _reduce_chunk(rows_v.at[buf_a], (zeros, zeros, zeros, zeros))

        @pl.when(i < CPW // 4 - 1)
        def _():
            start(4 * i + 4 + buf_a, buf_a)

        wait(buf_b)
        acc = _reduce_chunk(rows_v.at[buf_b], acc)

        @pl.when(i < CPW // 4 - 1)
        def _():
            start(4 * i + 4 + buf_b, buf_b)

        a0, a1, a2, a3 = acc
        pooled_v[b, pl.ds(0, 16)] = a0 * INV_SEQ
        pooled_v[b, pl.ds(16, 16)] = a1 * INV_SEQ
        pooled_v[b, pl.ds(32, 16)] = a2 * INV_SEQ
        pooled_v[b, pl.ds(48, 16)] = a3 * INV_SEQ

    def body(i, carry):
        do_batch(i, 2 * i, 0, 1)
        do_batch(i, 2 * i + 1, 2, 3)
        return carry

    lax.fori_loop(0, CPW // 4, body, 0)

    pltpu.sync_copy(pooled_v, pooled_hbm.at[pl.ds(bbase, BPW)])


def _mm_body(p_ref, w_ref, b_ref, o_ref):
    o_ref[...] = (
        jnp.dot(p_ref[...], w_ref[...], preferred_element_type=jnp.float32)
        + b_ref[...]
    )


_mm = pl.pallas_call(
    _mm_body,
    out_shape=jax.ShapeDtypeStruct((BATCH, OUT), jnp.float32),
)


def kernel(text, emb_table, fc_w, fc_b):
    tt = text.astype(jnp.int32).T.reshape(BATCH * 2, CHUNK)
    pooled = _pooled_kernel(tt, emb_table)
    return _mm(pooled, fc_w, fc_b.reshape(1, OUT))
